# initial kernel scaffold (unmeasured)
import jax
import jax.numpy as jnp
from jax import lax
from jax.experimental import pallas as pl
from jax.experimental.pallas import tpu as pltpu


def kernel(
    x,
):
    def body(*refs):
        pass

    out_shape = jax.ShapeDtypeStruct(..., jnp.float32)
    return pl.pallas_call(body, out_shape=out_shape)(...)


# baseline (device time: 7682 ns/iter reference)
import jax
import jax.numpy as jnp
from jax import lax
from jax.experimental import pallas as pl
from jax.experimental.pallas import tpu as pltpu

N_DEV = 4


def kernel(x):
    m, n = x.shape

    def body(x_ref, out_ref, send_ref, recv_ref, send_sem, recv_sem):
        my = lax.axis_index("i")
        is_first = my == 0
        is_last = my == N_DEV - 1
        left = (my - 1) % N_DEV
        right = (my + 1) % N_DEV

        barrier_sem = pltpu.get_barrier_semaphore()
        for nbr in (left, right):
            pl.semaphore_signal(
                barrier_sem, inc=1,
                device_id=(nbr,), device_id_type=pl.DeviceIdType.MESH,
            )
        pl.semaphore_wait(barrier_sem, 2)

        v = x_ref[:, :]
        shift = 1
        while shift < m:
            v = v * jnp.concatenate(
                [jnp.ones((shift, n), v.dtype), v[: m - shift, :]], axis=0
            )
            shift *= 2

        @pl.when(is_first)
        def _():
            recv_ref[:, :] = jnp.ones((1, n), v.dtype)

        rdma = pltpu.make_async_remote_copy(
            src_ref=send_ref,
            dst_ref=recv_ref,
            send_sem=send_sem,
            recv_sem=recv_sem,
            device_id=(right,),
            device_id_type=pl.DeviceIdType.MESH,
        )

        @pl.when(~is_first)
        def _():
            rdma.wait_recv()

        @pl.when(~is_last)
        def _():
            send_ref[:, :] = recv_ref[:, :] * v[m - 1 : m, :]
            rdma.start()
            rdma.wait_send()

        out_ref[:, :] = v * recv_ref[:, :]

    return pl.pallas_call(
        body,
        out_shape=jax.ShapeDtypeStruct((m, n), x.dtype),
        in_specs=[pl.BlockSpec(memory_space=pltpu.VMEM)],
        out_specs=pl.BlockSpec(memory_space=pltpu.VMEM),
        scratch_shapes=[
            pltpu.VMEM((1, n), x.dtype),
            pltpu.VMEM((1, n), x.dtype),
            pltpu.SemaphoreType.DMA,
            pltpu.SemaphoreType.DMA,
        ],
        compiler_params=pltpu.CompilerParams(collective_id=0),
    )(x)


# device time: 6649 ns/iter; 1.1554x vs baseline; 1.1554x over previous
import jax
import jax.numpy as jnp
from jax import lax
from jax.experimental import pallas as pl
from jax.experimental.pallas import tpu as pltpu

N_DEV = 4


def kernel(x):
    m, n = x.shape

    def body(x_ref, out_ref, buf_ref, send_sems, recv_sems):
        my = lax.axis_index("i")

        barrier_sem = pltpu.get_barrier_semaphore()
        for k in range(1, N_DEV):
            pl.semaphore_signal(
                barrier_sem, inc=1,
                device_id=((my + k) % N_DEV,),
                device_id_type=pl.DeviceIdType.MESH,
            )
        pl.semaphore_wait(barrier_sem, N_DEV - 1)

        t = x_ref[:, :]
        size = m
        while size > 1:
            half = size // 2
            t = t[:half, :] * t[half:size, :]
            size = half
        buf_ref[0, :, :] = t

        rdmas = []
        for k in range(1, N_DEV):
            rdma = pltpu.make_async_remote_copy(
                src_ref=buf_ref.at[0],
                dst_ref=buf_ref.at[k],
                send_sem=send_sems.at[k],
                recv_sem=recv_sems.at[k],
                device_id=((my + k) % N_DEV,),
                device_id_type=pl.DeviceIdType.MESH,
            )
            rdma.start()
            rdmas.append(rdma)

        v = x_ref[:, :]
        shift = 1
        while shift < m:
            v = v * jnp.concatenate(
                [jnp.ones((shift, n), v.dtype), v[: m - shift, :]], axis=0
            )
            shift *= 2

        for rdma in rdmas:
            rdma.wait_recv()
        for rdma in rdmas:
            rdma.wait_send()

        iota_k = lax.broadcasted_iota(jnp.int32, (N_DEV, 1, n), 0)
        src = (my - iota_k) % N_DEV
        w = jnp.where(src < my, buf_ref[:, :, :], jnp.ones((), x_ref.dtype))
        e = w[0] * w[1] * w[2] * w[3]

        out_ref[:, :] = v * e

    return pl.pallas_call(
        body,
        out_shape=jax.ShapeDtypeStruct((m, n), x.dtype),
        in_specs=[pl.BlockSpec(memory_space=pltpu.VMEM)],
        out_specs=pl.BlockSpec(memory_space=pltpu.VMEM),
        scratch_shapes=[
            pltpu.VMEM((N_DEV, 1, n), x.dtype),
            pltpu.SemaphoreType.DMA((N_DEV,)),
            pltpu.SemaphoreType.DMA((N_DEV,)),
        ],
        compiler_params=pltpu.CompilerParams(collective_id=0),
    )(x)


# device time: 6038 ns/iter; 1.2723x vs baseline; 1.1012x over previous
import jax
import jax.numpy as jnp
from jax import lax
from jax.experimental import pallas as pl
from jax.experimental.pallas import tpu as pltpu

N_DEV = 4


def kernel(x):
    m, n = x.shape

    def body(x_ref, out_ref, buf_ref, send_sems, recv_sems):
        my = lax.axis_index("i")

        barrier_sem = pltpu.get_barrier_semaphore()
        for k in range(1, N_DEV):
            @pl.when(k <= my)
            def _():
                pl.semaphore_signal(
                    barrier_sem, inc=1,
                    device_id=((my - k) % N_DEV,),
                    device_id_type=pl.DeviceIdType.MESH,
                )

        t = x_ref[:, :]
        size = m
        while size > 1:
            half = size // 2
            t = t[:half, :] * t[half:size, :]
            size = half
        buf_ref[0, :, :] = t

        for d in range(N_DEV - 1):
            @pl.when(my == d)
            def _():
                pl.semaphore_wait(barrier_sem, N_DEV - 1 - d)

        rdmas = []
        for k in range(1, N_DEV):
            rdma = pltpu.make_async_remote_copy(
                src_ref=buf_ref.at[0],
                dst_ref=buf_ref.at[k],
                send_sem=send_sems.at[k],
                recv_sem=recv_sems.at[k],
                device_id=((my + k) % N_DEV,),
                device_id_type=pl.DeviceIdType.MESH,
            )
            rdmas.append(rdma)

            @pl.when(my + k < N_DEV)
            def _():
                rdma.start()

        v = x_ref[:, :]
        shift = 1
        while shift < m:
            v = v * jnp.concatenate(
                [jnp.ones((shift, n), v.dtype), v[: m - shift, :]], axis=0
            )
            shift *= 2

        for k, rdma in enumerate(rdmas, start=1):
            @pl.when(k <= my)
            def _():
                rdma.wait_recv()

        iota_k = lax.broadcasted_iota(jnp.int32, (N_DEV, 1, n), 0)
        mask = (iota_k >= 1) & (iota_k <= my)
        w = jnp.where(mask, buf_ref[:, :, :], jnp.ones((), x_ref.dtype))
        e = w[0] * w[1] * w[2] * w[3]

        out_ref[:, :] = v * e

        for k, rdma in enumerate(rdmas, start=1):
            @pl.when(my + k < N_DEV)
            def _():
                rdma.wait_send()

    return pl.pallas_call(
        body,
        out_shape=jax.ShapeDtypeStruct((m, n), x.dtype),
        in_specs=[pl.BlockSpec(memory_space=pltpu.VMEM)],
        out_specs=pl.BlockSpec(memory_space=pltpu.VMEM),
        scratch_shapes=[
            pltpu.VMEM((N_DEV, 1, n), x.dtype),
            pltpu.SemaphoreType.DMA((N_DEV,)),
            pltpu.SemaphoreType.DMA((N_DEV,)),
        ],
        compiler_params=pltpu.CompilerParams(collective_id=0),
    )(x)
